# Initial kernel scaffold; baseline (speedup 1.0000x reference)
import jax
import jax.numpy as jnp
from jax.experimental import pallas as pl


def kernel(x, userVecs, itemVecs, tagUserVecs, tagItemVecs):
    raise NotImplementedError("write your pallas kernel here")


# speedup vs baseline: 15.8502x; 15.8502x over previous
import functools

import jax
import jax.numpy as jnp
from jax import lax
from jax.experimental import pallas as pl
from jax.experimental.pallas import tpu as pltpu
from jax.experimental.pallas import tpu_sc as plsc

K = 32
M = 200
MP = 208
CH = 104
GAMMA = 0.6
NC = 2
NS = 16
NW = NC * NS
L = 16


def _sc_time_attn(uid, iid, tid, nid, tag_pad, delta_pad,
                  userVecs, itemVecs, tagUserVecs, tagItemVecs):
    B = uid.shape[0]
    CB = B // NW
    mesh = plsc.VectorSubcoreMesh(core_axis_name="c", subcore_axis_name="s")

    @functools.partial(
        pl.kernel,
        mesh=mesh,
        out_type=jax.ShapeDtypeStruct((B // L, L), jnp.float32),
        compiler_params=pltpu.CompilerParams(use_tc_tiling_on_sc=False),
        scratch_types=[
            pltpu.VMEM((CB, 2, CH), jnp.int32),
            pltpu.VMEM((CB, MP), jnp.int32),
            pltpu.VMEM((CB,), jnp.int32),
            pltpu.VMEM((CB,), jnp.int32),
            pltpu.VMEM((CB,), jnp.int32),
            pltpu.VMEM((CB,), jnp.int32),
            pltpu.VMEM((CB, K), jnp.float32),
            pltpu.VMEM((CB, K), jnp.float32),
            pltpu.VMEM((CB, K), jnp.float32),
            pltpu.VMEM((CB, K), jnp.float32),
            pltpu.VMEM((CB, K), jnp.float32),
            pltpu.VMEM((CB, K), jnp.float32),
            pltpu.VMEM((MP, K), jnp.float32),
            pltpu.VMEM((CB // L, L), jnp.float32),
            pltpu.SemaphoreType.DMA,
            pltpu.SemaphoreType.DMA,
        ],
    )
    def body(uid_h, iid_h, tid_h, nid_h, tag_h, delta_h,
             uV_h, iV_h, tuV_h, tiV_h, out_h,
             tag_v, delta_v, uid_v, iid_v, tid_v, nid_v,
             uvec, ivec, utag, itag, nutag, nitag, rows, r_v,
             sem0, sem1):
        wid = lax.axis_index("s") * NC + lax.axis_index("c")
        base = wid * CB

        pltpu.sync_copy(tag_h.at[pl.ds(base, CB)], tag_v)
        pltpu.sync_copy(delta_h.at[pl.ds(base, CB)], delta_v)
        pltpu.sync_copy(uid_h.at[pl.ds(base, CB)], uid_v)
        pltpu.sync_copy(iid_h.at[pl.ds(base, CB)], iid_v)
        pltpu.sync_copy(tid_h.at[pl.ds(base, CB)], tid_v)
        pltpu.sync_copy(nid_h.at[pl.ds(base, CB)], nid_v)

        cps = [
            pltpu.async_copy(uV_h.at[uid_v], uvec, sem0),
            pltpu.async_copy(iV_h.at[iid_v], ivec, sem0),
            pltpu.async_copy(tuV_h.at[tid_v], utag, sem0),
            pltpu.async_copy(tiV_h.at[tid_v], itag, sem0),
            pltpu.async_copy(tuV_h.at[nid_v], nutag, sem0),
            pltpu.async_copy(tiV_h.at[nid_v], nitag, sem0),
        ]
        for cp in cps:
            cp.wait()

        lane = lax.iota(jnp.int32, L)

        def allsum(v):
            for s in (8, 4, 2, 1):
                v = v + v.at[lane ^ s].get(mode="promise_in_bounds")
            return v

        def sample_body(b, r_carry):
            g0 = pltpu.async_copy(tuV_h.at[tag_v.at[b, 0]],
                                  rows.at[pl.ds(0, CH)], sem0)
            g1 = pltpu.async_copy(tuV_h.at[tag_v.at[b, 1]],
                                  rows.at[pl.ds(CH, CH)], sem1)

            avs = []
            ssum = jnp.zeros((L,), jnp.float32)
            for c in range(MP // L):
                d = delta_v[b, pl.ds(c * L, L)].astype(jnp.float32)
                a = jnp.exp(d * -0.5)
                avs.append(a)
                ssum = ssum + a
            inv = 1.0 / allsum(ssum)

            g0.wait()
            g1.wait()

            h0 = jnp.zeros((L,), jnp.float32)
            h1 = jnp.zeros((L,), jnp.float32)
            for c in range(MP // L):
                a = avs[c]
                for j in range(L):
                    m = c * L + j
                    w = a[j]
                    h0 = h0 + w * rows[m, pl.ds(0, L)]
                    h1 = h1 + w * rows[m, pl.ds(L, L)]

            gi = GAMMA * inv
            mix0 = (1.0 - GAMMA) * uvec[b, pl.ds(0, L)] + gi * h0
            mix1 = (1.0 - GAMMA) * uvec[b, pl.ds(L, L)] + gi * h1
            d0 = utag[b, pl.ds(0, L)] - nutag[b, pl.ds(0, L)]
            d1 = utag[b, pl.ds(L, L)] - nutag[b, pl.ds(L, L)]
            e0 = itag[b, pl.ds(0, L)] - nitag[b, pl.ds(0, L)]
            e1 = itag[b, pl.ds(L, L)] - nitag[b, pl.ds(L, L)]
            acc = (mix0 * d0 + mix1 * d1
                   + ivec[b, pl.ds(0, L)] * e0 + ivec[b, pl.ds(L, L)] * e1)
            r = allsum(acc)
            r_carry = jnp.where(lane == (b & (L - 1)), r, r_carry)
            r_v[b >> 4, pl.ds(0, L)] = r_carry
            return r_carry

        lax.fori_loop(0, CB, sample_body, jnp.zeros((L,), jnp.float32))
        pltpu.sync_copy(r_v, out_h.at[pl.ds(wid * (CB // L), CB // L)])

    return body(uid, iid, tid, nid, tag_pad, delta_pad,
                userVecs, itemVecs, tagUserVecs, tagItemVecs)


def kernel(x, userVecs, itemVecs, tagUserVecs, tagItemVecs):
    B = x.shape[0]
    uid = x[:, 0]
    iid = x[:, 1]
    tid = x[:, 2]
    nid = x[:, 3]
    tag_mem = x[:, 4:4 + M]
    now = x[:, 4 + M]
    time_mem = x[:, -M:]
    tag_pad = jnp.pad(tag_mem, ((0, 0), (0, MP - M))).reshape(B, 2, CH)
    delta = now[:, None] - time_mem
    delta_pad = jnp.pad(delta, ((0, 0), (0, MP - M)),
                        constant_values=1_000_000_000)
    out = _sc_time_attn(uid, iid, tid, nid, tag_pad, delta_pad,
                        userVecs, itemVecs, tagUserVecs, tagItemVecs)
    return out.reshape(B)
